# baseline (speedup 1.0000x reference)
import functools

import jax
import jax.numpy as jnp
from jax import lax
from jax.experimental import pallas as pl
from jax.experimental.pallas import tpu as pltpu
from jax.experimental.pallas import tpu_sc as plsc

N = 10000
D = 128
E = 320000
MIN_NORM = 1e-15
PROJ_EPS = 4e-3

NC = 2
NS = 16
NW = NC * NS
L = 16

B = 80
EPT0 = 10240
EPT1 = 10240
EP = NS * (EPT0 + EPT1)
NCH0 = EPT0 // B
NCH1 = EPT1 // B
NP = 10240
RPS = NP // NS


def _logmap_body(x_ref, o_ref):
    x = x_ref[...]
    nrm = jnp.maximum(jnp.sqrt(jnp.sum(x * x, axis=1, keepdims=True)), MIN_NORM)
    t = jnp.clip(nrm, -1.0 + 1e-5, 1.0 - 1e-5)
    at = 0.5 * jnp.log((1.0 + t) / (1.0 - t))
    o_ref[...] = x * (at / nrm)


def _logmap(x):
    return pl.pallas_call(
        _logmap_body,
        grid=(10,),
        in_specs=[pl.BlockSpec((N // 10, D), lambda i: (i, 0))],
        out_specs=pl.BlockSpec((N // 10, D), lambda i: (i, 0)),
        out_shape=jax.ShapeDtypeStruct((N, D), jnp.float32),
    )(x)


NBUF = 4
UNROLL = 4


def _spmm_body(xt_hbm, row_hbm, col_hbm, val_hbm, out_hbm,
               rowv, colv, valv, msgs, acc, *sems):
    rsems = sems[0:NBUF]
    csems = sems[NBUF:2 * NBUF]
    vsems = sems[2 * NBUF:3 * NBUF]
    gsems = sems[3 * NBUF:4 * NBUF]
    ssems = sems[4 * NBUF:5 * NBUF]
    cid = lax.axis_index("c")
    sid = lax.axis_index("s")

    nch = jnp.where(cid == 0, NCH0, NCH1)
    ebase = jnp.where(cid == 0, sid * EPT0, NS * EPT0 + sid * EPT1)

    def zrow(r, c2):
        for k in range(D // L):
            msgs[0, r, pl.ds(k * L, L)] = jnp.zeros((L,), jnp.float32)
        return c2

    lax.fori_loop(0, B, zrow, 0)
    for j in range(RPS // 80):
        pltpu.sync_copy(msgs.at[0, pl.ds(0, 80)],
                        acc.at[pl.ds(sid * RPS + j * 80, 80)])
    plsc.subcore_barrier()

    def idx_start(ci, s):
        base = ebase + ci * B
        pltpu.async_copy(row_hbm.at[pl.ds(base, B)], rowv.at[s], rsems[s])
        pltpu.async_copy(col_hbm.at[pl.ds(base, B)], colv.at[s], csems[s])
        pltpu.async_copy(val_hbm.at[pl.ds(base, B)], valv.at[s], vsems[s])

    def col_wait(ci, s):
        base = ebase + ci * B
        pltpu.make_async_copy(col_hbm.at[pl.ds(base, B)], colv.at[s],
                              csems[s]).wait()

    def rowval_wait(ci, s):
        base = ebase + ci * B
        pltpu.make_async_copy(row_hbm.at[pl.ds(base, B)], rowv.at[s],
                              rsems[s]).wait()
        pltpu.make_async_copy(val_hbm.at[pl.ds(base, B)], valv.at[s],
                              vsems[s]).wait()

    def gather_start(ci, s):
        pltpu.async_copy(xt_hbm.at[colv.at[s]], msgs.at[s], gsems[s])

    def gather_wait(ci, s):
        pltpu.make_async_copy(xt_hbm.at[colv.at[s]], msgs.at[s],
                              gsems[s]).wait()

    def scatter_start(ci, s):
        pltpu.async_copy(msgs.at[s], acc.at[rowv.at[s]], ssems[s], add=True)

    def scatter_wait(ci, s):
        pltpu.make_async_copy(msgs.at[s], acc.at[rowv.at[s]],
                              ssems[s]).wait()

    idx_start(0, 0)
    idx_start(1, 1)
    col_wait(0, 0)
    gather_start(0, 0)

    def outer(g, carry):
        for s in range(NBUF):
            ci = g * NBUF + s
            gather_wait(ci, s)
            rowval_wait(ci, s)

            @pl.when(ci >= 2)
            def _():
                scatter_wait(ci - 2, (s - 2) % NBUF)

            @pl.when(ci + 2 < nch)
            def _():
                idx_start(ci + 2, (s + 2) % NBUF)

            @pl.when(ci + 1 < nch)
            def _():
                col_wait(ci + 1, (s + 1) % NBUF)
                gather_start(ci + 1, (s + 1) % NBUF)

            def sgroup(g, c2):
                vv = valv[s, pl.ds(g * L, L)]
                for j in range(L):
                    v = lax.gather(
                        vv, jnp.full((L, 1), j, jnp.int32),
                        lax.GatherDimensionNumbers(
                            offset_dims=(), collapsed_slice_dims=(0,),
                            start_index_map=(0,)),
                        slice_sizes=(1,),
                        mode=lax.GatherScatterMode.PROMISE_IN_BOUNDS)
                    b0 = g * L + j
                    for k in range(D // L):
                        msgs[s, b0, pl.ds(k * L, L)] = (
                            msgs[s, b0, pl.ds(k * L, L)] * v)
                return c2

            lax.fori_loop(0, B // L, sgroup, 0)
            scatter_start(ci, s)
        return carry

    lax.fori_loop(0, nch // NBUF, outer, 0)
    scatter_wait(0, NBUF - 2)
    scatter_wait(0, NBUF - 1)
    plsc.subcore_barrier()

    pltpu.sync_copy(acc.at[pl.ds(sid * RPS, RPS)],
                    out_hbm.at[pl.ds(cid * NP + sid * RPS, RPS)])


def _spmm(xt, rows, cols, vals):
    mesh = plsc.VectorSubcoreMesh(core_axis_name="c", subcore_axis_name="s")
    f = functools.partial(
        pl.kernel,
        mesh=mesh,
        out_type=jax.ShapeDtypeStruct((2 * NP, D), jnp.float32),
        scratch_types=[
            pltpu.VMEM((NBUF, B), jnp.int32),
            pltpu.VMEM((NBUF, B), jnp.int32),
            pltpu.VMEM((NBUF, B), jnp.float32),
            pltpu.VMEM((NBUF, B, D), jnp.float32),
            pltpu.VMEM_SHARED((NP, D), jnp.float32),
        ] + [pltpu.SemaphoreType.DMA] * (5 * NBUF),
    )(_spmm_body)
    return f(xt, rows, cols, vals)


def _post_body(a_ref, b_ref, o_ref):
    u = a_ref[...] + b_ref[...]
    un = jnp.maximum(jnp.sqrt(jnp.sum(u * u, axis=1, keepdims=True)), MIN_NORM)
    y = jnp.tanh(un) * (u / un)
    yn = jnp.maximum(jnp.sqrt(jnp.sum(y * y, axis=1, keepdims=True)), MIN_NORM)
    maxnorm = 1.0 - PROJ_EPS
    o_ref[...] = jnp.where(yn > maxnorm, y / yn * maxnorm, y)


def _post(parts):
    blk = 80
    return pl.pallas_call(
        _post_body,
        grid=(N // blk,),
        in_specs=[pl.BlockSpec((blk, D), lambda i: (i, 0)),
                  pl.BlockSpec((blk, D), lambda i: (i + NP // blk, 0))],
        out_specs=pl.BlockSpec((blk, D), lambda i: (i, 0)),
        out_shape=jax.ShapeDtypeStruct((N, D), jnp.float32),
    )(parts, parts)


def kernel(x, adj_indices, adj_values):
    xt = _logmap(x)
    pad = EP - E
    fill = jnp.arange(pad, dtype=jnp.int32) % N
    rows = jnp.concatenate([adj_indices[0], fill])
    cols = jnp.concatenate([adj_indices[1], fill])
    vals = jnp.pad(adj_values, (0, pad))
    parts = _spmm(xt, rows, cols, vals)
    return _post(parts)


# speedup vs baseline: 1.2483x; 1.2483x over previous
import functools

import jax
import jax.numpy as jnp
from jax import lax
from jax.experimental import pallas as pl
from jax.experimental.pallas import tpu as pltpu
from jax.experimental.pallas import tpu_sc as plsc

N = 10000
D = 128
E = 320000
MIN_NORM = 1e-15
PROJ_EPS = 4e-3

NC = 2
NS = 16
NW = NC * NS
L = 16

B = 80
EPT0 = 10240
EPT1 = 10240
EP = NS * (EPT0 + EPT1)
NCH0 = EPT0 // B
NCH1 = EPT1 // B
NP = 10240
RPS = NP // NS


def _logmap_body(x_ref, o_ref):
    x = x_ref[...]
    nrm = jnp.maximum(jnp.sqrt(jnp.sum(x * x, axis=1, keepdims=True)), MIN_NORM)
    t = jnp.clip(nrm, -1.0 + 1e-5, 1.0 - 1e-5)
    at = 0.5 * jnp.log((1.0 + t) / (1.0 - t))
    o_ref[...] = x * (at / nrm)


def _logmap(x):
    return pl.pallas_call(
        _logmap_body,
        grid=(10,),
        in_specs=[pl.BlockSpec((N // 10, D), lambda i: (i, 0))],
        out_specs=pl.BlockSpec((N // 10, D), lambda i: (i, 0)),
        out_shape=jax.ShapeDtypeStruct((N, D), jnp.float32),
    )(x)


NBUF = 4
UNROLL = 4


def _spmm_body(xt_hbm, row_hbm, col_hbm, val_hbm, out_hbm,
               rowv, colv, valv, msgs, acc, *sems):
    rsems = sems[0:NBUF]
    csems = sems[NBUF:2 * NBUF]
    vsems = sems[2 * NBUF:3 * NBUF]
    gsems = sems[3 * NBUF:4 * NBUF]
    ssems = sems[4 * NBUF:5 * NBUF]
    cid = lax.axis_index("c")
    sid = lax.axis_index("s")

    nch = jnp.where(cid == 0, NCH0, NCH1)
    ebase = jnp.where(cid == 0, sid * EPT0, NS * EPT0 + sid * EPT1)

    def zrow(r, c2):
        for k in range(D // L):
            msgs[0, r, pl.ds(k * L, L)] = jnp.zeros((L,), jnp.float32)
        return c2

    lax.fori_loop(0, B, zrow, 0)
    for j in range(RPS // B):
        pltpu.sync_copy(msgs.at[0], acc.at[pl.ds(sid * RPS + j * B, B)])
    plsc.subcore_barrier()

    def idx_start(ci, s):
        base = ebase + ci * B
        pltpu.async_copy(row_hbm.at[pl.ds(base, B)], rowv.at[s], rsems[s])
        pltpu.async_copy(col_hbm.at[pl.ds(base, B)], colv.at[s], csems[s])
        pltpu.async_copy(val_hbm.at[pl.ds(base, B)], valv.at[s], vsems[s])

    def col_wait(ci, s):
        base = ebase + ci * B
        pltpu.make_async_copy(col_hbm.at[pl.ds(base, B)], colv.at[s],
                              csems[s]).wait()

    def rowval_wait(ci, s):
        base = ebase + ci * B
        pltpu.make_async_copy(row_hbm.at[pl.ds(base, B)], rowv.at[s],
                              rsems[s]).wait()
        pltpu.make_async_copy(val_hbm.at[pl.ds(base, B)], valv.at[s],
                              vsems[s]).wait()

    def gather_start(ci, s):
        pltpu.async_copy(xt_hbm.at[colv.at[s]], msgs.at[s], gsems[s])

    def gather_wait(ci, s):
        pltpu.make_async_copy(xt_hbm.at[colv.at[s]], msgs.at[s],
                              gsems[s]).wait()

    def scatter_start(ci, s):
        pltpu.async_copy(msgs.at[s], acc.at[rowv.at[s]], ssems[s], add=True)

    def scatter_wait(ci, s):
        pltpu.make_async_copy(msgs.at[s], acc.at[rowv.at[s]],
                              ssems[s]).wait()

    idx_start(0, 0)
    idx_start(1, 1)
    col_wait(0, 0)
    gather_start(0, 0)

    def outer(g, carry):
        for s in range(NBUF):
            ci = g * NBUF + s
            gather_wait(ci, s)
            rowval_wait(ci, s)

            @pl.when(ci >= 2)
            def _():
                scatter_wait(ci - 2, (s - 2) % NBUF)

            @pl.when(ci + 2 < nch)
            def _():
                idx_start(ci + 2, (s + 2) % NBUF)

            @pl.when(ci + 1 < nch)
            def _():
                col_wait(ci + 1, (s + 1) % NBUF)
                gather_start(ci + 1, (s + 1) % NBUF)

            def sgroup(g, c2):
                vv = valv[s, pl.ds(g * L, L)]
                for j in range(L):
                    v = lax.gather(
                        vv, jnp.full((L, 1), j, jnp.int32),
                        lax.GatherDimensionNumbers(
                            offset_dims=(), collapsed_slice_dims=(0,),
                            start_index_map=(0,)),
                        slice_sizes=(1,),
                        mode=lax.GatherScatterMode.PROMISE_IN_BOUNDS)
                    b0 = g * L + j
                    for k in range(D // L):
                        msgs[s, b0, pl.ds(k * L, L)] = (
                            msgs[s, b0, pl.ds(k * L, L)] * v)
                return c2

            lax.fori_loop(0, B // L, sgroup, 0)
            scatter_start(ci, s)
        return carry

    lax.fori_loop(0, nch // NBUF, outer, 0)
    scatter_wait(0, NBUF - 2)
    scatter_wait(0, NBUF - 1)
    plsc.subcore_barrier()

    pltpu.sync_copy(acc.at[pl.ds(sid * RPS, RPS)],
                    out_hbm.at[pl.ds(cid * NP + sid * RPS, RPS)])


def _spmm(xt, rows, cols, vals):
    mesh = plsc.VectorSubcoreMesh(core_axis_name="c", subcore_axis_name="s")
    f = functools.partial(
        pl.kernel,
        mesh=mesh,
        out_type=jax.ShapeDtypeStruct((2 * NP, D), jnp.float32),
        scratch_types=[
            pltpu.VMEM((NBUF, B), jnp.int32),
            pltpu.VMEM((NBUF, B), jnp.int32),
            pltpu.VMEM((NBUF, B), jnp.float32),
            pltpu.VMEM((NBUF, B, D), jnp.float32),
            pltpu.VMEM_SHARED((NP, D), jnp.float32),
        ] + [pltpu.SemaphoreType.DMA] * (5 * NBUF),
    )(_spmm_body)
    return f(xt, rows, cols, vals)


def _post_body(a_ref, b_ref, o_ref):
    u = a_ref[...] + b_ref[...]
    un = jnp.maximum(jnp.sqrt(jnp.sum(u * u, axis=1, keepdims=True)), MIN_NORM)
    y = jnp.tanh(un) * (u / un)
    yn = jnp.maximum(jnp.sqrt(jnp.sum(y * y, axis=1, keepdims=True)), MIN_NORM)
    maxnorm = 1.0 - PROJ_EPS
    o_ref[...] = jnp.where(yn > maxnorm, y / yn * maxnorm, y)


def _post(a, b):
    return pl.pallas_call(
        _post_body,
        grid=(10,),
        in_specs=[pl.BlockSpec((N // 10, D), lambda i: (i, 0)),
                  pl.BlockSpec((N // 10, D), lambda i: (i, 0))],
        out_specs=pl.BlockSpec((N // 10, D), lambda i: (i, 0)),
        out_shape=jax.ShapeDtypeStruct((N, D), jnp.float32),
    )(a, b)


def kernel(x, adj_indices, adj_values):
    xt = _logmap(x)
    pad = EP - E
    fill = jnp.arange(pad, dtype=jnp.int32) % N
    rows = jnp.concatenate([adj_indices[0], fill])
    cols = jnp.concatenate([adj_indices[1], fill])
    vals = jnp.pad(adj_values, (0, pad))
    parts = _spmm(xt, rows, cols, vals)
    return _post(parts[:N], parts[NP:NP + N])
